# baseline (speedup 1.0000x reference)
import jax
import jax.numpy as jnp
from jax.experimental import pallas as pl

_N = 10000
_NP = 10240
_RB = 2048
_D = 256


def _proj_body(h_ref, wt_ref, a_ref, z_ref, st_ref):
    zb = jnp.dot(h_ref[...], wt_ref[...], preferred_element_type=jnp.float32)
    z_ref[...] = zb
    st_ref[...] = jax.lax.dot_general(
        a_ref[...], zb, (((1,), (1,)), ((), ())),
        preferred_element_type=jnp.float32)


def _project(h_pad, wt, a2):
    z, st = pl.pallas_call(
        _proj_body,
        grid=(_NP // _RB,),
        in_specs=[
            pl.BlockSpec((_RB, _D), lambda i: (i, 0)),
            pl.BlockSpec((_D, _D), lambda i: (0, 0)),
            pl.BlockSpec((2, _D), lambda i: (0, 0)),
        ],
        out_specs=[
            pl.BlockSpec((_RB, _D), lambda i: (i, 0)),
            pl.BlockSpec((2, _RB), lambda i: (0, i)),
        ],
        out_shape=[
            jax.ShapeDtypeStruct((_NP, _D), jnp.float32),
            jax.ShapeDtypeStruct((2, _NP), jnp.float32),
        ],
    )(h_pad, wt, a2)
    return z, st


def kernel(h, edge_index, W_fc, W_attn):
    n, d = h.shape
    h_pad = jnp.pad(h, ((0, _NP - n), (0, 0)))
    wt = W_fc.T
    a2 = W_attn.reshape(2, d)
    z_pad, st = _project(h_pad, wt, a2)
    z = z_pad[:n]
    s = st[0, :n]
    t = st[1, :n]

    src = edge_index[0]
    dst = edge_index[1]
    e = s[src] + t[dst]
    e = jnp.where(e < 0, 0.01 * e, e)
    w = jnp.exp(e)
    denom = jax.ops.segment_sum(w, dst, num_segments=n)
    alpha = w / denom[dst]
    out = jax.ops.segment_sum(alpha[:, None] * z[src], dst, num_segments=n)
    return out


# speedup vs baseline: 3.8398x; 3.8398x over previous
import dataclasses
import functools

import jax
import jax.numpy as jnp
from jax import lax
from jax.experimental import pallas as pl
from jax.experimental.pallas import tpu as pltpu
from jax.experimental.pallas import tpu_sc as plsc

_N = 10000
_NP = 10240
_RB = 2048
_D = 256
_DQ = 64
_NPASS = 2
_E = 160000
_NT = 16
_EPT = _E // _NT
_G = _EPT // 16
_RPT = _NP // _NT
_ZR = 128


def _proj_body(h_ref, wt_ref, a_ref, z_ref, st_ref):
    zb = jnp.dot(h_ref[...], wt_ref[...], preferred_element_type=jnp.float32)
    for q in range(4):
        z_ref[q] = zb[:, q * _DQ:(q + 1) * _DQ]
    st_ref[...] = lax.dot_general(
        a_ref[...], zb, (((1,), (1,)), ((), ())),
        preferred_element_type=jnp.float32)


def _project(h_pad, wt, a2):
    return pl.pallas_call(
        _proj_body,
        grid=(_NP // _RB,),
        in_specs=[
            pl.BlockSpec((_RB, _D), lambda i: (i, 0)),
            pl.BlockSpec((_D, _D), lambda i: (0, 0)),
            pl.BlockSpec((2, _D), lambda i: (0, 0)),
        ],
        out_specs=[
            pl.BlockSpec((4, _RB, _DQ), lambda i: (0, i, 0)),
            pl.BlockSpec((2, _RB), lambda i: (0, i)),
        ],
        out_shape=[
            jax.ShapeDtypeStruct((4, _NP, _DQ), jnp.float32),
            jax.ShapeDtypeStruct((2, _NP), jnp.float32),
        ],
    )(h_pad, wt, a2)


def _sc_compiler_params():
    cp = pltpu.CompilerParams()
    if "needs_layout_passes" in pltpu.CompilerParams.__dataclass_fields__:
        cp = dataclasses.replace(cp, needs_layout_passes=False)
    if "use_tc_tiling_on_sc" in pltpu.CompilerParams.__dataclass_fields__:
        cp = dataclasses.replace(cp, use_tc_tiling_on_sc=False)
    return cp


def _sc_edge(z_split, st, src3, dst3):
    mesh = plsc.VectorSubcoreMesh(core_axis_name="c", subcore_axis_name="s")

    @functools.partial(
        pl.kernel,
        compiler_params=_sc_compiler_params(),
        out_type=[
            jax.ShapeDtypeStruct((4, _NP, _DQ), jnp.float32),
            jax.ShapeDtypeStruct((_NT, _NP // 16, 16), jnp.float32),
        ],
        mesh=mesh,
        scratch_types=[
            pltpu.VMEM((_G, 16), jnp.int32),
            pltpu.VMEM((_G, 16), jnp.int32),
            pltpu.VMEM((_NP,), jnp.float32),
            pltpu.VMEM((_NP,), jnp.float32),
            pltpu.VMEM((_NP // 16, 16), jnp.float32),
            pltpu.VMEM((16, _DQ), jnp.float32),
            pltpu.VMEM((_ZR, _DQ), jnp.float32),
            pltpu.VMEM_SHARED((_NP, _DQ), jnp.float32),
        ],
    )
    def k(z_hbm, st_hbm, src_hbm, dst_hbm, acc_hbm, dnp_hbm,
          srcv, dstv, sv, tv, dnv, rowsv, zv, accs):
        cid = lax.axis_index("c")
        sid = lax.axis_index("s")

        pltpu.sync_copy(src_hbm.at[sid], srcv)
        pltpu.sync_copy(dst_hbm.at[sid], dstv)
        pltpu.sync_copy(st_hbm.at[0], sv)
        pltpu.sync_copy(st_hbm.at[1], tv)

        zvec = jnp.zeros((16,), jnp.float32)

        @pl.loop(0, _ZR)
        def _(i):
            @pl.loop(0, _DQ, step=16)
            def _(q):
                zv[i, pl.ds(q, 16)] = zvec

        @pl.loop(0, _NP // 16)
        def _(i):
            dnv[i] = zvec

        for p in range(_NPASS):
            zc = z_hbm.at[cid * _NPASS + p]
            for b in range(_RPT // _ZR):
                pltpu.sync_copy(zv, accs.at[pl.ds(sid * _RPT + b * _ZR, _ZR)])
            plsc.subcore_barrier()

            @pl.loop(0, _G)
            def _(g):
                s16 = srcv[g]
                d16 = dstv[g]
                sg = plsc.load_gather(sv, [s16])
                tg = plsc.load_gather(tv, [d16])
                e = sg + tg
                e = jnp.where(e < 0.0, e * jnp.float32(0.01), e)
                w = jnp.exp(e)

                if p == 0:
                    @pl.when(cid == 0)
                    def _():
                        plsc.addupdate_scatter(
                            dnv, [lax.shift_right_logical(d16, 4), d16 & 15], w)

                pltpu.sync_copy(zc.at[srcv.at[g]], rowsv)
                for j in range(16):
                    wj = jnp.full((16,), w[j], jnp.float32)
                    for q in range(_DQ // 16):
                        rowsv[j, pl.ds(q * 16, 16)] = (
                            rowsv[j, pl.ds(q * 16, 16)] * wj)
                pltpu.sync_copy(rowsv, accs.at[dstv.at[g]], add=True)

            plsc.subcore_barrier()
            pltpu.sync_copy(
                accs.at[pl.ds(sid * _RPT, _RPT)],
                acc_hbm.at[cid * _NPASS + p].at[pl.ds(sid * _RPT, _RPT)])

        @pl.when(cid == 0)
        def _():
            pltpu.sync_copy(dnv, dnp_hbm.at[sid])

    return k(z_split, st, src3, dst3)


def _epi_body(acc_ref, dn_ref, o_ref):
    dsum = jnp.sum(dn_ref[...], axis=0)
    cat = jnp.concatenate([acc_ref[q] for q in range(4)], axis=1)
    d = dsum[:, None]
    o_ref[...] = jnp.where(d > 0.0, cat / d, jnp.float32(0.0))


def _epilogue(acc, dnp):
    return pl.pallas_call(
        _epi_body,
        grid=(_NP // _RB,),
        in_specs=[
            pl.BlockSpec((4, _RB, _DQ), lambda i: (0, i, 0)),
            pl.BlockSpec((_NT, _RB), lambda i: (0, i)),
        ],
        out_specs=pl.BlockSpec((_RB, _D), lambda i: (i, 0)),
        out_shape=jax.ShapeDtypeStruct((_NP, _D), jnp.float32),
    )(acc, dnp)


def kernel(h, edge_index, W_fc, W_attn):
    n, d = h.shape
    h_pad = jnp.pad(h, ((0, _NP - n), (0, 0)))
    z_split, st = _project(h_pad, W_fc.T, W_attn.reshape(2, d))
    src3 = edge_index[0].reshape(_NT, _G, 16)
    dst3 = edge_index[1].reshape(_NT, _G, 16)
    acc, dnp = _sc_edge(z_split, st, src3, dst3)
    out = _epilogue(acc, dnp.reshape(_NT, _NP))
    return out[:n]


# speedup vs baseline: 10.8478x; 2.8251x over previous
import dataclasses
import functools

import jax
import jax.numpy as jnp
from jax import lax
from jax.experimental import pallas as pl
from jax.experimental.pallas import tpu as pltpu
from jax.experimental.pallas import tpu_sc as plsc

_N = 10000
_NP = 10240
_RB = 2048
_D = 256
_DQ = 64
_NPASS = 2
_E = 160000
_NT = 16
_EPT = _E // _NT
_G = _EPT // 16
_RPT = _NP // _NT
_ZR = 128
_NB = 4
_GM = _G - 1


def _proj_body(h_ref, wt_ref, a_ref, z_ref, st_ref):
    zb = jnp.dot(h_ref[...], wt_ref[...], preferred_element_type=jnp.float32)
    for q in range(4):
        z_ref[q] = zb[:, q * _DQ:(q + 1) * _DQ]
    st_ref[...] = lax.dot_general(
        a_ref[...], zb, (((1,), (1,)), ((), ())),
        preferred_element_type=jnp.float32)


def _project(h_pad, wt, a2):
    return pl.pallas_call(
        _proj_body,
        grid=(_NP // _RB,),
        in_specs=[
            pl.BlockSpec((_RB, _D), lambda i: (i, 0)),
            pl.BlockSpec((_D, _D), lambda i: (0, 0)),
            pl.BlockSpec((2, _D), lambda i: (0, 0)),
        ],
        out_specs=[
            pl.BlockSpec((4, _RB, _DQ), lambda i: (0, i, 0)),
            pl.BlockSpec((2, _RB), lambda i: (0, i)),
        ],
        out_shape=[
            jax.ShapeDtypeStruct((4, _NP, _DQ), jnp.float32),
            jax.ShapeDtypeStruct((2, _NP), jnp.float32),
        ],
    )(h_pad, wt, a2)


def _sc_compiler_params():
    cp = pltpu.CompilerParams()
    if "needs_layout_passes" in pltpu.CompilerParams.__dataclass_fields__:
        cp = dataclasses.replace(cp, needs_layout_passes=False)
    if "use_tc_tiling_on_sc" in pltpu.CompilerParams.__dataclass_fields__:
        cp = dataclasses.replace(cp, use_tc_tiling_on_sc=False)
    return cp


def _sc_edge(z_split, st, src3, dst3):
    mesh = plsc.VectorSubcoreMesh(core_axis_name="c", subcore_axis_name="s")

    @functools.partial(
        pl.kernel,
        compiler_params=_sc_compiler_params(),
        out_type=[
            jax.ShapeDtypeStruct((4, _NP, _DQ), jnp.float32),
            jax.ShapeDtypeStruct((_NT, _NP // 16, 16), jnp.float32),
        ],
        mesh=mesh,
        scratch_types=[
            pltpu.VMEM((_G, 16), jnp.int32),
            pltpu.VMEM((_G, 16), jnp.int32),
            pltpu.VMEM((_NP,), jnp.float32),
            pltpu.VMEM((_NP,), jnp.float32),
            pltpu.VMEM((_NP // 16, 16), jnp.float32),
            pltpu.VMEM((_NB, 16, _DQ), jnp.float32),
            pltpu.VMEM((_G, 16), jnp.float32),
            pltpu.VMEM((_ZR, _DQ), jnp.float32),
            pltpu.VMEM_SHARED((_NP, _DQ), jnp.float32),
        ] + [pltpu.SemaphoreType.DMA] * (2 * _NB),
    )
    def k(z_hbm, st_hbm, src_hbm, dst_hbm, acc_hbm, dnp_hbm,
          srcv, dstv, sv, tv, dnv, rows, wbuf, zv, accs, *sems):
        gsems = sems[:_NB]
        ssems = sems[_NB:]
        cid = lax.axis_index("c")
        sid = lax.axis_index("s")

        pltpu.sync_copy(src_hbm.at[sid], srcv)
        pltpu.sync_copy(dst_hbm.at[sid], dstv)
        pltpu.sync_copy(st_hbm.at[0], sv)
        pltpu.sync_copy(st_hbm.at[1], tv)

        zvec = jnp.zeros((16,), jnp.float32)

        @pl.loop(0, _ZR)
        def _(i):
            @pl.loop(0, _DQ, step=16)
            def _(q):
                zv[i, pl.ds(q, 16)] = zvec

        @pl.loop(0, _NP // 16)
        def _(i):
            dnv[i] = zvec

        for p in range(_NPASS):
            zc = z_hbm.at[cid * _NPASS + p]
            for b in range(_RPT // _ZR):
                pltpu.sync_copy(zv, accs.at[pl.ds(sid * _RPT + b * _ZR, _ZR)])
            plsc.subcore_barrier()

            def scalars(g):
                if p == 0:
                    s16 = srcv[g]
                    d16 = dstv[g]
                    sg = plsc.load_gather(sv, [s16])
                    tg = plsc.load_gather(tv, [d16])
                    e = sg + tg
                    e = jnp.where(e < 0.0, e * jnp.float32(0.01), e)
                    w = jnp.exp(e)
                    wbuf[g] = w

                    @pl.when(cid == 0)
                    def _():
                        plsc.addupdate_scatter(
                            dnv, [lax.shift_right_logical(d16, 4), d16 & 15], w)
                    return w
                return wbuf[g]

            def issue_gather(g, b):
                pltpu.async_copy(zc.at[srcv.at[g]], rows.at[b], gsems[b])

            def wait_gather(b):
                pltpu.make_async_copy(
                    zc.at[pl.ds(0, 16)], rows.at[b], gsems[b]).wait()

            def issue_scatter(g, b):
                pltpu.async_copy(
                    rows.at[b], accs.at[dstv.at[g]], ssems[b], add=True)

            def wait_scatter(b):
                pltpu.make_async_copy(
                    rows.at[b], accs.at[pl.ds(0, 16)], ssems[b]).wait()

            def scale(g, b, w):
                for j in range(16):
                    wj = jnp.full((16,), w[j], jnp.float32)
                    for q in range(_DQ // 16):
                        rows[b, j, pl.ds(q * 16, 16)] = (
                            rows[b, j, pl.ds(q * 16, 16)] * wj)

            for b in range(_NB - 1):
                issue_gather(b, b)

            @pl.loop(0, _GM // _NB)
            def _(i):
                g0 = i * _NB
                for b in range(_NB):
                    g = g0 + b
                    w = scalars(g)
                    bprev = (b - 1) % _NB

                    @pl.when(jnp.logical_and(g > 0, g + _NB - 1 < _G))
                    def _():
                        wait_scatter(bprev)
                        issue_gather(g + _NB - 1, bprev)

                    if b == 0:
                        @pl.when(g == 0)
                        def _():
                            issue_gather(_NB - 1, _NB - 1)

                    wait_gather(b)
                    scale(g, b, w)
                    issue_scatter(g, b)

            w = scalars(_G - 1)
            wait_gather(0)
            scale(_G - 1, 0, w)
            issue_scatter(_G - 1, 0)
            for b in range(1, _NB):
                wait_scatter(b)
            wait_scatter(0)

            plsc.subcore_barrier()
            pltpu.sync_copy(
                accs.at[pl.ds(sid * _RPT, _RPT)],
                acc_hbm.at[cid * _NPASS + p].at[pl.ds(sid * _RPT, _RPT)])

        @pl.when(cid == 0)
        def _():
            pltpu.sync_copy(dnv, dnp_hbm.at[sid])

    return k(z_split, st, src3, dst3)


def _epi_body(acc_ref, dn_ref, o_ref):
    dsum = jnp.sum(dn_ref[...], axis=0)
    cat = jnp.concatenate([acc_ref[q] for q in range(4)], axis=1)
    d = dsum[:, None]
    o_ref[...] = jnp.where(d > 0.0, cat / d, jnp.float32(0.0))


def _epilogue(acc, dnp):
    return pl.pallas_call(
        _epi_body,
        grid=(_NP // _RB,),
        in_specs=[
            pl.BlockSpec((4, _RB, _DQ), lambda i: (0, i, 0)),
            pl.BlockSpec((_NT, _RB), lambda i: (0, i)),
        ],
        out_specs=pl.BlockSpec((_RB, _D), lambda i: (i, 0)),
        out_shape=jax.ShapeDtypeStruct((_NP, _D), jnp.float32),
    )(acc, dnp)


def kernel(h, edge_index, W_fc, W_attn):
    n, d = h.shape
    h_pad = jnp.pad(h, ((0, _NP - n), (0, 0)))
    z_split, st = _project(h_pad, W_fc.T, W_attn.reshape(2, d))
    src3 = edge_index[0].reshape(_NT, _G, 16)
    dst3 = edge_index[1].reshape(_NT, _G, 16)
    acc, dnp = _sc_edge(z_split, st, src3, dst3)
    out = _epilogue(acc, dnp.reshape(_NT, _NP))
    return out[:n]


# speedup vs baseline: 12.7422x; 1.1746x over previous
import dataclasses
import functools

import jax
import jax.numpy as jnp
from jax import lax
from jax.experimental import pallas as pl
from jax.experimental.pallas import tpu as pltpu
from jax.experimental.pallas import tpu_sc as plsc

_N = 10000
_NP = 10240
_RB = 2048
_D = 256
_DQ = 64
_NPASS = 2
_E = 160000
_NT = 16
_EPT = _E // _NT
_G = _EPT // 16
_RPT = _NP // _NT
_ZR = 128
_NB = 6
_GM = _G - 1


def _proj_body(h_ref, wt_ref, a_ref, z_ref, st_ref):
    zb = jnp.dot(h_ref[...], wt_ref[...], preferred_element_type=jnp.float32)
    for q in range(4):
        z_ref[q] = zb[:, q * _DQ:(q + 1) * _DQ]
    st_ref[...] = lax.dot_general(
        a_ref[...], zb, (((1,), (1,)), ((), ())),
        preferred_element_type=jnp.float32)


def _project(h_pad, wt, a2):
    return pl.pallas_call(
        _proj_body,
        grid=(_NP // _RB,),
        in_specs=[
            pl.BlockSpec((_RB, _D), lambda i: (i, 0)),
            pl.BlockSpec((_D, _D), lambda i: (0, 0)),
            pl.BlockSpec((2, _D), lambda i: (0, 0)),
        ],
        out_specs=[
            pl.BlockSpec((4, _RB, _DQ), lambda i: (0, i, 0)),
            pl.BlockSpec((2, _RB), lambda i: (0, i)),
        ],
        out_shape=[
            jax.ShapeDtypeStruct((4, _NP, _DQ), jnp.float32),
            jax.ShapeDtypeStruct((2, _NP), jnp.float32),
        ],
    )(h_pad, wt, a2)


def _sc_compiler_params():
    cp = pltpu.CompilerParams()
    if "needs_layout_passes" in pltpu.CompilerParams.__dataclass_fields__:
        cp = dataclasses.replace(cp, needs_layout_passes=False)
    if "use_tc_tiling_on_sc" in pltpu.CompilerParams.__dataclass_fields__:
        cp = dataclasses.replace(cp, use_tc_tiling_on_sc=False)
    return cp


def _sc_edge(z_split, st, src3, dst3):
    mesh = plsc.VectorSubcoreMesh(core_axis_name="c", subcore_axis_name="s")

    @functools.partial(
        pl.kernel,
        compiler_params=_sc_compiler_params(),
        out_type=[
            jax.ShapeDtypeStruct((4, _NP, _DQ), jnp.float32),
            jax.ShapeDtypeStruct((_NT, _NP // 16, 16), jnp.float32),
        ],
        mesh=mesh,
        scratch_types=[
            pltpu.VMEM((_G, 16), jnp.int32),
            pltpu.VMEM((_G, 16), jnp.int32),
            pltpu.VMEM((_NP,), jnp.float32),
            pltpu.VMEM((_NP,), jnp.float32),
            pltpu.VMEM((_NP // 16, 16), jnp.float32),
            pltpu.VMEM((_NB, 16, _DQ), jnp.float32),
            pltpu.VMEM((_G, 16), jnp.float32),
            pltpu.VMEM((_ZR, _DQ), jnp.float32),
            pltpu.VMEM_SHARED((_NP, _DQ), jnp.float32),
        ] + [pltpu.SemaphoreType.DMA] * (2 * _NB),
    )
    def k(z_hbm, st_hbm, src_hbm, dst_hbm, acc_hbm, dnp_hbm,
          srcv, dstv, sv, tv, dnv, rows, wbuf, zv, accs, *sems):
        gsems = sems[:_NB]
        ssems = sems[_NB:]
        cid = lax.axis_index("c")
        sid = lax.axis_index("s")

        pltpu.sync_copy(src_hbm.at[sid], srcv)
        pltpu.sync_copy(dst_hbm.at[sid], dstv)
        pltpu.sync_copy(st_hbm.at[0], sv)
        pltpu.sync_copy(st_hbm.at[1], tv)

        zvec = jnp.zeros((16,), jnp.float32)

        @pl.loop(0, _ZR)
        def _(i):
            @pl.loop(0, _DQ, step=16)
            def _(q):
                zv[i, pl.ds(q, 16)] = zvec

        @pl.loop(0, _NP // 16)
        def _(i):
            dnv[i] = zvec

        for p in range(_NPASS):
            zc = z_hbm.at[cid * _NPASS + p]
            for b in range(_RPT // _ZR):
                pltpu.sync_copy(zv, accs.at[pl.ds(sid * _RPT + b * _ZR, _ZR)])
            plsc.subcore_barrier()

            def scalars(g):
                if p == 0:
                    s16 = srcv[g]
                    d16 = dstv[g]
                    sg = plsc.load_gather(sv, [s16])
                    tg = plsc.load_gather(tv, [d16])
                    e = sg + tg
                    e = jnp.where(e < 0.0, e * jnp.float32(0.01), e)
                    w = jnp.exp(e)
                    wbuf[g] = w

                    @pl.when(cid == 0)
                    def _():
                        plsc.addupdate_scatter(
                            dnv, [lax.shift_right_logical(d16, 4), d16 & 15], w)
                    return w
                return wbuf[g]

            def issue_gather(g, b):
                pltpu.async_copy(zc.at[srcv.at[g]], rows.at[b], gsems[b])

            def wait_gather(b):
                pltpu.make_async_copy(
                    zc.at[pl.ds(0, 16)], rows.at[b], gsems[b]).wait()

            def issue_scatter(g, b):
                pltpu.async_copy(
                    rows.at[b], accs.at[dstv.at[g]], ssems[b], add=True)

            def wait_scatter(b):
                pltpu.make_async_copy(
                    rows.at[b], accs.at[pl.ds(0, 16)], ssems[b]).wait()

            def scale(g, b, w):
                for j in range(16):
                    wj = jnp.full((16,), w[j], jnp.float32)
                    for q in range(_DQ // 16):
                        rows[b, j, pl.ds(q * 16, 16)] = (
                            rows[b, j, pl.ds(q * 16, 16)] * wj)

            for b in range(_NB - 1):
                issue_gather(b, b)

            @pl.loop(0, _GM // _NB)
            def _(i):
                g0 = i * _NB
                for b in range(_NB):
                    g = g0 + b
                    w = scalars(g)
                    bprev = (b - 1) % _NB

                    @pl.when(jnp.logical_and(g > 0, g + _NB - 1 < _G))
                    def _():
                        wait_scatter(bprev)
                        issue_gather(g + _NB - 1, bprev)

                    if b == 0:
                        @pl.when(g == 0)
                        def _():
                            issue_gather(_NB - 1, _NB - 1)

                    wait_gather(b)
                    scale(g, b, w)
                    issue_scatter(g, b)

            w = scalars(_G - 1)
            wait_gather(0)
            scale(_G - 1, 0, w)
            issue_scatter(_G - 1, 0)
            for b in range(1, _NB):
                wait_scatter(b)
            wait_scatter(0)

            plsc.subcore_barrier()
            pltpu.sync_copy(
                accs.at[pl.ds(sid * _RPT, _RPT)],
                acc_hbm.at[cid * _NPASS + p].at[pl.ds(sid * _RPT, _RPT)])

        @pl.when(cid == 0)
        def _():
            pltpu.sync_copy(dnv, dnp_hbm.at[sid])

    return k(z_split, st, src3, dst3)


def _epi_body(acc_ref, dn_ref, o_ref):
    dsum = jnp.sum(dn_ref[...], axis=0)
    cat = jnp.concatenate([acc_ref[q] for q in range(4)], axis=1)
    d = dsum[:, None]
    o_ref[...] = jnp.where(d > 0.0, cat / d, jnp.float32(0.0))


def _epilogue(acc, dnp):
    return pl.pallas_call(
        _epi_body,
        grid=(_NP // _RB,),
        in_specs=[
            pl.BlockSpec((4, _RB, _DQ), lambda i: (0, i, 0)),
            pl.BlockSpec((_NT, _RB), lambda i: (0, i)),
        ],
        out_specs=pl.BlockSpec((_RB, _D), lambda i: (i, 0)),
        out_shape=jax.ShapeDtypeStruct((_NP, _D), jnp.float32),
    )(acc, dnp)


def kernel(h, edge_index, W_fc, W_attn):
    n, d = h.shape
    h_pad = jnp.pad(h, ((0, _NP - n), (0, 0)))
    z_split, st = _project(h_pad, W_fc.T, W_attn.reshape(2, d))
    src3 = edge_index[0].reshape(_NT, _G, 16)
    dst3 = edge_index[1].reshape(_NT, _G, 16)
    acc, dnp = _sc_edge(z_split, st, src3, dst3)
    out = _epilogue(acc, dnp.reshape(_NT, _NP))
    return out[:n]


# speedup vs baseline: 12.7978x; 1.0044x over previous
import dataclasses
import functools

import jax
import jax.numpy as jnp
from jax import lax
from jax.experimental import pallas as pl
from jax.experimental.pallas import tpu as pltpu
from jax.experimental.pallas import tpu_sc as plsc

_N = 10000
_NP = 10240
_RB = 2048
_D = 256
_DQ = 64
_NPASS = 2
_E = 160000
_NT = 16
_EPT = _E // _NT
_G = _EPT // 16
_RPT = _NP // _NT
_ZR = 128
_NB = 8
_GM = _G - 1


def _proj_body(h_ref, wt_ref, a_ref, z_ref, st_ref):
    zb = jnp.dot(h_ref[...], wt_ref[...], preferred_element_type=jnp.float32)
    for q in range(4):
        z_ref[q] = zb[:, q * _DQ:(q + 1) * _DQ]
    st_ref[...] = lax.dot_general(
        a_ref[...], zb, (((1,), (1,)), ((), ())),
        preferred_element_type=jnp.float32)


def _project(h_pad, wt, a2):
    return pl.pallas_call(
        _proj_body,
        grid=(_NP // _RB,),
        in_specs=[
            pl.BlockSpec((_RB, _D), lambda i: (i, 0)),
            pl.BlockSpec((_D, _D), lambda i: (0, 0)),
            pl.BlockSpec((2, _D), lambda i: (0, 0)),
        ],
        out_specs=[
            pl.BlockSpec((4, _RB, _DQ), lambda i: (0, i, 0)),
            pl.BlockSpec((2, _RB), lambda i: (0, i)),
        ],
        out_shape=[
            jax.ShapeDtypeStruct((4, _NP, _DQ), jnp.float32),
            jax.ShapeDtypeStruct((2, _NP), jnp.float32),
        ],
    )(h_pad, wt, a2)


def _sc_compiler_params():
    cp = pltpu.CompilerParams()
    if "needs_layout_passes" in pltpu.CompilerParams.__dataclass_fields__:
        cp = dataclasses.replace(cp, needs_layout_passes=False)
    if "use_tc_tiling_on_sc" in pltpu.CompilerParams.__dataclass_fields__:
        cp = dataclasses.replace(cp, use_tc_tiling_on_sc=False)
    return cp


def _sc_edge(z_split, st, src3, dst3):
    mesh = plsc.VectorSubcoreMesh(core_axis_name="c", subcore_axis_name="s")

    @functools.partial(
        pl.kernel,
        compiler_params=_sc_compiler_params(),
        out_type=[
            jax.ShapeDtypeStruct((4, _NP, _DQ), jnp.float32),
            jax.ShapeDtypeStruct((_NT, _NP // 16, 16), jnp.float32),
        ],
        mesh=mesh,
        scratch_types=[
            pltpu.VMEM((_G, 16), jnp.int32),
            pltpu.VMEM((_G, 16), jnp.int32),
            pltpu.VMEM((_NP,), jnp.float32),
            pltpu.VMEM((_NP,), jnp.float32),
            pltpu.VMEM((_NP // 16, 16), jnp.float32),
            pltpu.VMEM((_NB, 16, _DQ), jnp.float32),
            pltpu.VMEM((_G, 16), jnp.float32),
            pltpu.VMEM((_ZR, _DQ), jnp.float32),
            pltpu.VMEM_SHARED((_NP, _DQ), jnp.float32),
        ] + [pltpu.SemaphoreType.DMA] * (2 * _NB),
    )
    def k(z_hbm, st_hbm, src_hbm, dst_hbm, acc_hbm, dnp_hbm,
          srcv, dstv, sv, tv, dnv, rows, wbuf, zv, accs, *sems):
        gsems = sems[:_NB]
        ssems = sems[_NB:]
        cid = lax.axis_index("c")
        sid = lax.axis_index("s")

        pltpu.sync_copy(src_hbm.at[sid], srcv)
        pltpu.sync_copy(dst_hbm.at[sid], dstv)
        pltpu.sync_copy(st_hbm.at[0], sv)
        pltpu.sync_copy(st_hbm.at[1], tv)

        zvec = jnp.zeros((16,), jnp.float32)

        @pl.loop(0, _ZR)
        def _(i):
            @pl.loop(0, _DQ, step=16)
            def _(q):
                zv[i, pl.ds(q, 16)] = zvec

        @pl.loop(0, _NP // 16)
        def _(i):
            dnv[i] = zvec

        for p in range(_NPASS):
            zc = z_hbm.at[cid * _NPASS + p]
            for b in range(_RPT // _ZR):
                pltpu.sync_copy(zv, accs.at[pl.ds(sid * _RPT + b * _ZR, _ZR)])
            plsc.subcore_barrier()

            def scalars(g):
                if p == 0:
                    s16 = srcv[g]
                    d16 = dstv[g]
                    sg = plsc.load_gather(sv, [s16])
                    tg = plsc.load_gather(tv, [d16])
                    e = sg + tg
                    e = jnp.where(e < 0.0, e * jnp.float32(0.01), e)
                    w = jnp.exp(e)
                    wbuf[g] = w

                    @pl.when(cid == 0)
                    def _():
                        plsc.addupdate_scatter(
                            dnv, [lax.shift_right_logical(d16, 4), d16 & 15], w)
                    return w
                return wbuf[g]

            def issue_gather(g, b):
                pltpu.async_copy(zc.at[srcv.at[g]], rows.at[b], gsems[b])

            def wait_gather(b):
                pltpu.make_async_copy(
                    zc.at[pl.ds(0, 16)], rows.at[b], gsems[b]).wait()

            def issue_scatter(g, b):
                pltpu.async_copy(
                    rows.at[b], accs.at[dstv.at[g]], ssems[b], add=True)

            def wait_scatter(b):
                pltpu.make_async_copy(
                    rows.at[b], accs.at[pl.ds(0, 16)], ssems[b]).wait()

            def scale(g, b, w):
                for j in range(16):
                    wj = jnp.full((16,), w[j], jnp.float32)
                    for q in range(_DQ // 16):
                        rows[b, j, pl.ds(q * 16, 16)] = (
                            rows[b, j, pl.ds(q * 16, 16)] * wj)

            for b in range(_NB - 1):
                issue_gather(b, b)

            @pl.loop(0, _GM // _NB)
            def _(i):
                g0 = i * _NB
                for b in range(_NB):
                    g = g0 + b
                    w = scalars(g)
                    bprev = (b - 1) % _NB

                    @pl.when(jnp.logical_and(g > 0, g + _NB - 1 < _G))
                    def _():
                        wait_scatter(bprev)
                        issue_gather(g + _NB - 1, bprev)

                    if b == 0:
                        @pl.when(g == 0)
                        def _():
                            issue_gather(_NB - 1, _NB - 1)

                    wait_gather(b)
                    scale(g, b, w)
                    issue_scatter(g, b)

            w = scalars(_G - 1)
            wait_gather(0)
            scale(_G - 1, 0, w)
            issue_scatter(_G - 1, 0)
            for b in range(1, _NB):
                wait_scatter(b)
            wait_scatter(0)

            plsc.subcore_barrier()
            pltpu.sync_copy(
                accs.at[pl.ds(sid * _RPT, _RPT)],
                acc_hbm.at[cid * _NPASS + p].at[pl.ds(sid * _RPT, _RPT)])

        @pl.when(cid == 0)
        def _():
            pltpu.sync_copy(dnv, dnp_hbm.at[sid])

    return k(z_split, st, src3, dst3)


def _epi_body(acc_ref, dn_ref, o_ref):
    dsum = jnp.sum(dn_ref[...], axis=0)
    cat = jnp.concatenate([acc_ref[q] for q in range(4)], axis=1)
    d = dsum[:, None]
    o_ref[...] = jnp.where(d > 0.0, cat / d, jnp.float32(0.0))


def _epilogue(acc, dnp):
    return pl.pallas_call(
        _epi_body,
        grid=(_NP // _RB,),
        in_specs=[
            pl.BlockSpec((4, _RB, _DQ), lambda i: (0, i, 0)),
            pl.BlockSpec((_NT, _RB), lambda i: (0, i)),
        ],
        out_specs=pl.BlockSpec((_RB, _D), lambda i: (i, 0)),
        out_shape=jax.ShapeDtypeStruct((_NP, _D), jnp.float32),
    )(acc, dnp)


def kernel(h, edge_index, W_fc, W_attn):
    n, d = h.shape
    h_pad = jnp.pad(h, ((0, _NP - n), (0, 0)))
    z_split, st = _project(h_pad, W_fc.T, W_attn.reshape(2, d))
    src3 = edge_index[0].reshape(_NT, _G, 16)
    dst3 = edge_index[1].reshape(_NT, _G, 16)
    acc, dnp = _sc_edge(z_split, st, src3, dst3)
    out = _epilogue(acc, dnp.reshape(_NT, _NP))
    return out[:n]
